# Initial kernel scaffold; baseline (speedup 1.0000x reference)
import jax
import jax.numpy as jnp
from jax.experimental import pallas as pl


def kernel(x, mask, W_in, b_in, Wl1, Wr1, att1, bias1, Wl2, Wr2, att2, bias2, W_ih0, W_hh0, b_ih0, b_hh0, W_ih1, W_hh1, b_ih1, b_hh1):
    raise NotImplementedError("write your pallas kernel here")


# speedup vs baseline: 147.2205x; 147.2205x over previous
import functools
import jax
import jax.numpy as jnp
from jax.experimental import pallas as pl

_B, _T, _N, _F = 4, 12, 207, 2
_H, _HEADS = 64, 4
_NP = 208
_JB = 8
_NJB = _NP // _JB
_C = _HEADS * _H
_NEG = -1e30


def _proj1_kernel(x_ref, Wi_ref, bi_ref, Wl_ref, Wr_ref, a06_ref,
                  xl_ref, xr_ref, sl_ref, sr_ref):
    xb = x_ref[0, 0]
    xin = bi_ref[...]
    xin = (xb[:, 0:1] * Wi_ref[0:1, :] + xb[:, 1:2] * Wi_ref[1:2, :]) + xin
    _proj_common(xin, Wl_ref, Wr_ref, a06_ref, xl_ref, xr_ref, sl_ref, sr_ref)


def _proj2_kernel(h_ref, Wl_ref, Wr_ref, a06_ref,
                  xl_ref, xr_ref, sl_ref, sr_ref):
    _proj_common(h_ref[0, 0], Wl_ref, Wr_ref, a06_ref,
                 xl_ref, xr_ref, sl_ref, sr_ref)


def _proj_common(xin, Wl_ref, Wr_ref, a06_ref, xl_ref, xr_ref, sl_ref, sr_ref):
    xl = jnp.dot(xin, Wl_ref[...], preferred_element_type=jnp.float32)
    xr = jnp.dot(xin, Wr_ref[...], preferred_element_type=jnp.float32)
    sl = jnp.dot(xl, a06_ref[...], preferred_element_type=jnp.float32)
    sr = jnp.dot(xr, a06_ref[...], preferred_element_type=jnp.float32)
    row = jax.lax.broadcasted_iota(jnp.int32, (_NP, _HEADS), 0)
    sl = jnp.where(row < _N, sl, _NEG)
    xl_ref[0, 0] = xl
    xr_ref[0, 0] = xr
    sl_ref[0, 0] = sl
    sr_ref[0, 0] = sr


def _attn_kernel(xl_ref, sl_ref, xr_ref, sr_ref, a04_ref, bias_ref, out_ref):
    xl = xl_ref[0, 0]
    sl = sl_ref[0, 0]
    xrb = xr_ref[0, 0]
    srb = sr_ref[0, 0]
    absv = [jnp.abs(xl + xrb[k][None, :]) for k in range(_JB)]
    bigv = jnp.concatenate(absv, axis=0)
    p = jnp.dot(bigv, a04_ref[...], preferred_element_type=jnp.float32)
    e = p.reshape(_JB, _NP, _HEADS) + sl[None, :, :] + srb[:, None, :]
    mx = jnp.max(e, axis=1, keepdims=True)
    a = jnp.exp(e - mx)
    s = jnp.sum(a, axis=1, keepdims=True)
    a = a / s
    outs = []
    for k in range(_JB):
        Sk = jax.lax.dot_general(a[k], xl, (((0,), (0,)), ((), ())),
                                 preferred_element_type=jnp.float32)
        outs.append(Sk[None])
    S = jnp.concatenate(outs, axis=0)
    o = (S[:, 0, 0:64] + S[:, 1, 64:128] + S[:, 2, 128:192]
         + S[:, 3, 192:256]) * 0.25 + bias_ref[...]
    out_ref[0, 0] = jnp.where(o > 0, o, jnp.exp(jnp.minimum(o, 0.0)) - 1.0)


def _lstm_kernel(h_ref, Wx0_ref, Wh0_ref, b0_ref, Wx1_ref, Wh1_ref, b1_ref,
                 out_ref):
    BN = _B * _NP

    def gates(g):
        i = jax.nn.sigmoid(g[:, 0 * _H:1 * _H])
        f = jax.nn.sigmoid(g[:, 1 * _H:2 * _H])
        gg = jnp.tanh(g[:, 2 * _H:3 * _H])
        o = jax.nn.sigmoid(g[:, 3 * _H:4 * _H])
        return i, f, gg, o

    def step(t, carry):
        h1, c1, h2, c2 = carry
        xt = h_ref[t]
        g = (jnp.dot(xt, Wx0_ref[...], preferred_element_type=jnp.float32)
             + jnp.dot(h1, Wh0_ref[...], preferred_element_type=jnp.float32)
             + b0_ref[...])
        i, f, gg, o = gates(g)
        c1 = f * c1 + i * gg
        h1 = o * jnp.tanh(c1)
        g2 = (jnp.dot(h1, Wx1_ref[...], preferred_element_type=jnp.float32)
              + jnp.dot(h2, Wh1_ref[...], preferred_element_type=jnp.float32)
              + b1_ref[...])
        i2, f2, gg2, o2 = gates(g2)
        c2 = f2 * c2 + i2 * gg2
        h2 = o2 * jnp.tanh(c2)
        return h1, c1, h2, c2

    z = jnp.zeros((BN, _H), jnp.float32)
    _, _, h2, _ = jax.lax.fori_loop(0, _T, step, (z, z, z, z))
    out_ref[...] = h2


def _attbd(att, scale):
    return (scale * (jnp.eye(_HEADS, dtype=jnp.float32)[:, None, :]
                     * att[:, :, None])).reshape(_C, _HEADS)


def _gat_layer(feats, Wi, bi, Wl, Wr, att, bias):
    f32 = jnp.float32
    a06 = _attbd(att, 0.6)
    a04 = _attbd(att, 0.4)
    fin = feats.shape[-1]
    wspec = lambda shp: pl.BlockSpec(shp, lambda t, b: (0, 0))
    if Wi is not None:
        kern = _proj1_kernel
        extra = (Wi, bi.reshape(1, _H))
        extra_specs = [wspec((_F, _H)), wspec((1, _H))]
    else:
        kern = _proj2_kernel
        extra = ()
        extra_specs = []
    xl, xr, sl, sr = pl.pallas_call(
        kern,
        grid=(_T, _B),
        in_specs=[pl.BlockSpec((1, 1, _NP, fin), lambda t, b: (t, b, 0, 0))]
        + extra_specs
        + [wspec((_H, _C)), wspec((_H, _C)), wspec((_C, _HEADS))],
        out_specs=[
            pl.BlockSpec((1, 1, _NP, _C), lambda t, b: (t, b, 0, 0)),
            pl.BlockSpec((1, 1, _NP, _C), lambda t, b: (t, b, 0, 0)),
            pl.BlockSpec((1, 1, _NP, _HEADS), lambda t, b: (t, b, 0, 0)),
            pl.BlockSpec((1, 1, _NP, _HEADS), lambda t, b: (t, b, 0, 0)),
        ],
        out_shape=[
            jax.ShapeDtypeStruct((_T, _B, _NP, _C), f32),
            jax.ShapeDtypeStruct((_T, _B, _NP, _C), f32),
            jax.ShapeDtypeStruct((_T, _B, _NP, _HEADS), f32),
            jax.ShapeDtypeStruct((_T, _B, _NP, _HEADS), f32),
        ],
    )(feats, *extra, Wl, Wr, a06)

    out = pl.pallas_call(
        _attn_kernel,
        grid=(_T, _B, _NJB),
        in_specs=[
            pl.BlockSpec((1, 1, _NP, _C), lambda t, b, j: (t, b, 0, 0)),
            pl.BlockSpec((1, 1, _NP, _HEADS), lambda t, b, j: (t, b, 0, 0)),
            pl.BlockSpec((1, 1, _JB, _C), lambda t, b, j: (t, b, j, 0)),
            pl.BlockSpec((1, 1, _JB, _HEADS), lambda t, b, j: (t, b, j, 0)),
            pl.BlockSpec((_C, _HEADS), lambda t, b, j: (0, 0)),
            pl.BlockSpec((1, _H), lambda t, b, j: (0, 0)),
        ],
        out_specs=pl.BlockSpec((1, 1, _JB, _H), lambda t, b, j: (t, b, j, 0)),
        out_shape=jax.ShapeDtypeStruct((_T, _B, _NP, _H), f32),
    )(xl, sl, xr, sr, a04, bias.reshape(1, _H))
    return out


def kernel(x, mask, W_in, b_in, Wl1, Wr1, att1, bias1, Wl2, Wr2, att2, bias2,
           W_ih0, W_hh0, b_ih0, b_hh0, W_ih1, W_hh1, b_ih1, b_hh1):
    f32 = jnp.float32
    xp = jnp.transpose(x, (1, 0, 2, 3))
    xp = jnp.pad(xp, ((0, 0), (0, 0), (0, _NP - _N), (0, 0)))

    h1 = _gat_layer(xp, W_in, b_in, Wl1, Wr1, att1, bias1)
    h2 = _gat_layer(h1, None, None, Wl2, Wr2, att2, bias2)

    hseq = h2.reshape(_T, _B * _NP, _H)
    wspec = lambda shp: pl.BlockSpec(shp, lambda: tuple(0 for _ in shp))
    out = pl.pallas_call(
        _lstm_kernel,
        in_specs=[
            wspec((_T, _B * _NP, _H)),
            wspec((_H, 4 * _H)), wspec((_H, 4 * _H)), wspec((1, 4 * _H)),
            wspec((_H, 4 * _H)), wspec((_H, 4 * _H)), wspec((1, 4 * _H)),
        ],
        out_specs=wspec((_B * _NP, _H)),
        out_shape=jax.ShapeDtypeStruct((_B * _NP, _H), f32),
    )(hseq,
      W_ih0.T, W_hh0.T, (b_ih0 + b_hh0).reshape(1, 4 * _H),
      W_ih1.T, W_hh1.T, (b_ih1 + b_hh1).reshape(1, 4 * _H))

    return out.reshape(_B, _NP, _H)[:, :_N].reshape(_B * _N, _H)


# speedup vs baseline: 202.3807x; 1.3747x over previous
import jax
import jax.numpy as jnp
from jax.experimental import pallas as pl
from jax.experimental.pallas import tpu as pltpu

_B, _T, _N, _F = 4, 12, 207, 2
_H, _HEADS = 64, 4
_NP = 208
_CP = 256
_JB = 8
_NJB = _NP // _JB
_C = _HEADS * _H
_NEG = -1e30


def _gat_layer(xin, Wl, WlT, WrT, a06T, a04T, bias, E_ref, xr_ref):
    f32 = jnp.float32
    xinp = jnp.pad(xin, ((0, _CP - _NP), (0, 0)))
    xinT = xinp.T
    xl = jnp.dot(xin, Wl, preferred_element_type=f32)
    xlT = jnp.dot(WlT, xinT, preferred_element_type=f32)
    xrT = jnp.dot(WrT, xinT, preferred_element_type=f32)
    slT = jnp.dot(a06T, xlT, preferred_element_type=f32)
    lane = jax.lax.broadcasted_iota(jnp.int32, (_HEADS, _CP), 1)
    slT = jnp.where(lane < _N, slT, _NEG)

    xr_ref[...] = jnp.dot(xin, jnp.transpose(WrT), preferred_element_type=f32)

    def block(kb, carry):
        (xl_T,) = carry
        xrb = xr_ref[pl.ds(kb * _JB, _JB), :]
        xrTb = xrb.T
        pts = []
        for k in range(_JB):
            piece = jnp.abs(xl_T + xrTb[:, k:k + 1])
            pts.append(jnp.dot(a04T, piece, preferred_element_type=f32))
        blk = jnp.stack(pts, axis=1)
        E_ref[:, pl.ds(kb * _JB, _JB), :] = blk
        return carry

    jax.lax.fori_loop(0, _NJB, block, (xlT,), unroll=1)

    E = E_ref[...] + slT[:, None, :]
    mx = jnp.max(E, axis=2, keepdims=True)
    A = jnp.exp(E - mx)
    A = A / jnp.sum(A, axis=2, keepdims=True)
    xlp = jnp.pad(xl, ((0, _CP - _NP), (0, 0)))
    o = None
    for h in range(_HEADS):
        oh = jnp.dot(A[h], xlp[:, h * _H:(h + 1) * _H],
                     preferred_element_type=f32)
        o = oh if o is None else o + oh
    o = o * 0.25 + bias
    return jnp.where(o > 0, o, jnp.exp(jnp.minimum(o, 0.0)) - 1.0)


def _gat_kernel(x_ref, Wi_ref, bi_ref,
                Wl1_ref, WlT1_ref, WrT1_ref, a06T1_ref, a04T1_ref, b1_ref,
                Wl2_ref, WlT2_ref, WrT2_ref, a06T2_ref, a04T2_ref, b2_ref,
                out_ref, E_ref, xr_ref):
    xb = x_ref[0, 0]
    xin = (xb[:, 0:1] * Wi_ref[0:1, :] + xb[:, 1:2] * Wi_ref[1:2, :]
           + bi_ref[...])
    h = _gat_layer(xin, Wl1_ref[...], WlT1_ref[...], WrT1_ref[...],
                   a06T1_ref[...], a04T1_ref[...], b1_ref[...], E_ref, xr_ref)
    h = _gat_layer(h, Wl2_ref[...], WlT2_ref[...], WrT2_ref[...],
                   a06T2_ref[...], a04T2_ref[...], b2_ref[...], E_ref, xr_ref)
    out_ref[0, 0] = h


def _lstm_kernel(h_ref, Wx0_ref, Wh0_ref, b0_ref, Wx1_ref, Wh1_ref, b1_ref,
                 out_ref):
    BN = _B * _NP

    def gates(g):
        i = jax.nn.sigmoid(g[:, 0 * _H:1 * _H])
        f = jax.nn.sigmoid(g[:, 1 * _H:2 * _H])
        gg = jnp.tanh(g[:, 2 * _H:3 * _H])
        o = jax.nn.sigmoid(g[:, 3 * _H:4 * _H])
        return i, f, gg, o

    def step(t, carry):
        h1, c1, h2, c2 = carry
        xt = h_ref[t]
        g = (jnp.dot(xt, Wx0_ref[...], preferred_element_type=jnp.float32)
             + jnp.dot(h1, Wh0_ref[...], preferred_element_type=jnp.float32)
             + b0_ref[...])
        i, f, gg, o = gates(g)
        c1 = f * c1 + i * gg
        h1 = o * jnp.tanh(c1)
        g2 = (jnp.dot(h1, Wx1_ref[...], preferred_element_type=jnp.float32)
              + jnp.dot(h2, Wh1_ref[...], preferred_element_type=jnp.float32)
              + b1_ref[...])
        i2, f2, gg2, o2 = gates(g2)
        c2 = f2 * c2 + i2 * gg2
        h2 = o2 * jnp.tanh(c2)
        return h1, c1, h2, c2

    z = jnp.zeros((BN, _H), jnp.float32)
    _, _, h2, _ = jax.lax.fori_loop(0, _T, step, (z, z, z, z))
    out_ref[...] = h2


def _attbd(att, scale):
    return (scale * (jnp.eye(_HEADS, dtype=jnp.float32)[:, None, :]
                     * att[:, :, None])).reshape(_C, _HEADS).T


def kernel(x, mask, W_in, b_in, Wl1, Wr1, att1, bias1, Wl2, Wr2, att2, bias2,
           W_ih0, W_hh0, b_ih0, b_hh0, W_ih1, W_hh1, b_ih1, b_hh1):
    f32 = jnp.float32
    xp = jnp.transpose(x, (1, 0, 2, 3))
    xp = jnp.pad(xp, ((0, 0), (0, 0), (0, _NP - _N), (0, 0)))

    wspec = lambda shp: pl.BlockSpec(shp, lambda t, b: tuple(0 for _ in shp))
    h2 = pl.pallas_call(
        _gat_kernel,
        grid=(_T, _B),
        in_specs=[pl.BlockSpec((1, 1, _NP, _F), lambda t, b: (t, b, 0, 0)),
                  wspec((_F, _H)), wspec((1, _H)),
                  wspec((_H, _C)), wspec((_C, _H)), wspec((_C, _H)),
                  wspec((_HEADS, _CP)), wspec((_HEADS, _CP)), wspec((1, _H)),
                  wspec((_H, _C)), wspec((_C, _H)), wspec((_C, _H)),
                  wspec((_HEADS, _CP)), wspec((_HEADS, _CP)), wspec((1, _H))],
        out_specs=pl.BlockSpec((1, 1, _NP, _H), lambda t, b: (t, b, 0, 0)),
        out_shape=jax.ShapeDtypeStruct((_T, _B, _NP, _H), f32),
        scratch_shapes=[pltpu.VMEM((_HEADS, _NP, _CP), f32),
                        pltpu.VMEM((_NP, _C), f32)],
    )(xp, W_in, b_in.reshape(1, _H),
      Wl1, Wl1.T, Wr1.T, _attbd(att1, 0.6), _attbd(att1, 0.4),
      bias1.reshape(1, _H),
      Wl2, Wl2.T, Wr2.T, _attbd(att2, 0.6), _attbd(att2, 0.4),
      bias2.reshape(1, _H))

    hseq = h2.reshape(_T, _B * _NP, _H)
    nspec = lambda shp: pl.BlockSpec(shp, lambda: tuple(0 for _ in shp))
    out = pl.pallas_call(
        _lstm_kernel,
        in_specs=[
            nspec((_T, _B * _NP, _H)),
            nspec((_H, 4 * _H)), nspec((_H, 4 * _H)), nspec((1, 4 * _H)),
            nspec((_H, 4 * _H)), nspec((_H, 4 * _H)), nspec((1, 4 * _H)),
        ],
        out_specs=nspec((_B * _NP, _H)),
        out_shape=jax.ShapeDtypeStruct((_B * _NP, _H), f32),
    )(hseq,
      W_ih0.T, W_hh0.T, (b_ih0 + b_hh0).reshape(1, 4 * _H),
      W_ih1.T, W_hh1.T, (b_ih1 + b_hh1).reshape(1, 4 * _H))

    return out.reshape(_B, _NP, _H)[:, :_N].reshape(_B * _N, _H)


# speedup vs baseline: 272.9043x; 1.3485x over previous
import jax
import jax.numpy as jnp
from jax.experimental import pallas as pl
from jax.experimental.pallas import tpu as pltpu

_B, _T, _N, _F = 4, 12, 207, 2
_H, _HEADS = 64, 4
_NP = 208
_CP = 256
_JB = 8
_NJB = _NP // _JB
_C = _HEADS * _H
_NEG = -1e30


def _gat_layer(xin, Wl, WlT, WrT, a06T, a04T, bias, E_ref, xr_ref):
    f32 = jnp.float32
    xinp = jnp.pad(xin, ((0, _CP - _NP), (0, 0)))
    xinT = xinp.T
    xl = jnp.dot(xin, Wl, preferred_element_type=f32)
    xlT = jnp.dot(WlT, xinT, preferred_element_type=f32)
    xrT = jnp.dot(WrT, xinT, preferred_element_type=f32)
    slT = jnp.dot(a06T, xlT, preferred_element_type=f32)
    lane = jax.lax.broadcasted_iota(jnp.int32, (_HEADS, _CP), 1)
    slT = jnp.where(lane < _N, slT, _NEG)

    xr_ref[...] = jnp.dot(xin, jnp.transpose(WrT), preferred_element_type=f32)

    bf16 = jnp.bfloat16
    xlTb = xlT.astype(bf16)
    a04Tb = a04T.astype(bf16)

    def block(kb, carry):
        (xl_Tb,) = carry
        xrb = xr_ref[pl.ds(kb * _JB, _JB), :]
        xrTb = xrb.T.astype(bf16)
        pieces = [jnp.abs(xl_Tb + xrTb[:, k:k + 1]) for k in range(_JB)]
        bigU = jnp.concatenate(pieces, axis=1)
        pT = jnp.dot(a04Tb, bigU, preferred_element_type=f32)
        E_ref[:, pl.ds(kb * _JB, _JB), :] = pT.reshape(_HEADS, _JB, _CP)
        return carry

    jax.lax.fori_loop(0, _NJB, block, (xlTb,), unroll=1)

    E = E_ref[...] + slT[:, None, :]
    mx = jnp.max(E, axis=2, keepdims=True)
    A = jnp.exp(E - mx)
    rs = 0.25 / jnp.sum(A, axis=2)
    xlp = jnp.pad(xl, ((0, _CP - _NP), (0, 0)))
    o = None
    for h in range(_HEADS):
        oh = jnp.dot(A[h], xlp[:, h * _H:(h + 1) * _H],
                     preferred_element_type=f32)
        oh = oh * rs[h][:, None]
        o = oh if o is None else o + oh
    o = o + bias
    return jnp.where(o > 0, o, jnp.exp(jnp.minimum(o, 0.0)) - 1.0)


def _gat_kernel(x_ref, Wi_ref, bi_ref,
                Wl1_ref, WlT1_ref, WrT1_ref, a06T1_ref, a04T1_ref, b1_ref,
                Wl2_ref, WlT2_ref, WrT2_ref, a06T2_ref, a04T2_ref, b2_ref,
                out_ref, E_ref, xr_ref):
    xb = x_ref[0, 0]
    xin = (xb[:, 0:1] * Wi_ref[0:1, :] + xb[:, 1:2] * Wi_ref[1:2, :]
           + bi_ref[...])
    h = _gat_layer(xin, Wl1_ref[...], WlT1_ref[...], WrT1_ref[...],
                   a06T1_ref[...], a04T1_ref[...], b1_ref[...], E_ref, xr_ref)
    h = _gat_layer(h, Wl2_ref[...], WlT2_ref[...], WrT2_ref[...],
                   a06T2_ref[...], a04T2_ref[...], b2_ref[...], E_ref, xr_ref)
    out_ref[0, 0] = h


def _lstm_kernel(h_ref, Wx0_ref, Wh0_ref, b0_ref, Wx1_ref, Wh1_ref, b1_ref,
                 out_ref):
    BN = _B * _NP

    def gates(g):
        i = jax.nn.sigmoid(g[:, 0 * _H:1 * _H])
        f = jax.nn.sigmoid(g[:, 1 * _H:2 * _H])
        gg = jnp.tanh(g[:, 2 * _H:3 * _H])
        o = jax.nn.sigmoid(g[:, 3 * _H:4 * _H])
        return i, f, gg, o

    def step(t, carry):
        h1, c1, h2, c2 = carry
        xt = h_ref[t]
        g = (jnp.dot(xt, Wx0_ref[...], preferred_element_type=jnp.float32)
             + jnp.dot(h1, Wh0_ref[...], preferred_element_type=jnp.float32)
             + b0_ref[...])
        i, f, gg, o = gates(g)
        c1 = f * c1 + i * gg
        h1 = o * jnp.tanh(c1)
        g2 = (jnp.dot(h1, Wx1_ref[...], preferred_element_type=jnp.float32)
              + jnp.dot(h2, Wh1_ref[...], preferred_element_type=jnp.float32)
              + b1_ref[...])
        i2, f2, gg2, o2 = gates(g2)
        c2 = f2 * c2 + i2 * gg2
        h2 = o2 * jnp.tanh(c2)
        return h1, c1, h2, c2

    z = jnp.zeros((BN, _H), jnp.float32)
    _, _, h2, _ = jax.lax.fori_loop(0, _T, step, (z, z, z, z))
    out_ref[...] = h2


def _attbd(att, scale):
    return (scale * (jnp.eye(_HEADS, dtype=jnp.float32)[:, None, :]
                     * att[:, :, None])).reshape(_C, _HEADS).T


def kernel(x, mask, W_in, b_in, Wl1, Wr1, att1, bias1, Wl2, Wr2, att2, bias2,
           W_ih0, W_hh0, b_ih0, b_hh0, W_ih1, W_hh1, b_ih1, b_hh1):
    f32 = jnp.float32
    xp = jnp.transpose(x, (1, 0, 2, 3))
    xp = jnp.pad(xp, ((0, 0), (0, 0), (0, _NP - _N), (0, 0)))

    wspec = lambda shp: pl.BlockSpec(shp, lambda t, b: tuple(0 for _ in shp))
    h2 = pl.pallas_call(
        _gat_kernel,
        grid=(_T, _B),
        in_specs=[pl.BlockSpec((1, 1, _NP, _F), lambda t, b: (t, b, 0, 0)),
                  wspec((_F, _H)), wspec((1, _H)),
                  wspec((_H, _C)), wspec((_C, _H)), wspec((_C, _H)),
                  wspec((_HEADS, _CP)), wspec((_HEADS, _CP)), wspec((1, _H)),
                  wspec((_H, _C)), wspec((_C, _H)), wspec((_C, _H)),
                  wspec((_HEADS, _CP)), wspec((_HEADS, _CP)), wspec((1, _H))],
        out_specs=pl.BlockSpec((1, 1, _NP, _H), lambda t, b: (t, b, 0, 0)),
        out_shape=jax.ShapeDtypeStruct((_T, _B, _NP, _H), f32),
        scratch_shapes=[pltpu.VMEM((_HEADS, _NP, _CP), f32),
                        pltpu.VMEM((_NP, _C), f32)],
    )(xp, W_in, b_in.reshape(1, _H),
      Wl1, Wl1.T, Wr1.T, _attbd(att1, 0.6), _attbd(att1, 0.4),
      bias1.reshape(1, _H),
      Wl2, Wl2.T, Wr2.T, _attbd(att2, 0.6), _attbd(att2, 0.4),
      bias2.reshape(1, _H))

    hseq = h2.reshape(_T, _B * _NP, _H)
    nspec = lambda shp: pl.BlockSpec(shp, lambda: tuple(0 for _ in shp))
    out = pl.pallas_call(
        _lstm_kernel,
        in_specs=[
            nspec((_T, _B * _NP, _H)),
            nspec((_H, 4 * _H)), nspec((_H, 4 * _H)), nspec((1, 4 * _H)),
            nspec((_H, 4 * _H)), nspec((_H, 4 * _H)), nspec((1, 4 * _H)),
        ],
        out_specs=nspec((_B * _NP, _H)),
        out_shape=jax.ShapeDtypeStruct((_B * _NP, _H), f32),
    )(hseq,
      W_ih0.T, W_hh0.T, (b_ih0 + b_hh0).reshape(1, 4 * _H),
      W_ih1.T, W_hh1.T, (b_ih1 + b_hh1).reshape(1, 4 * _H))

    return out.reshape(_B, _NP, _H)[:, :_N].reshape(_B * _N, _H)
